# baseline (speedup 1.0000x reference)
import functools

import jax
import jax.numpy as jnp
from jax import lax
from jax.experimental import pallas as pl
from jax.experimental.pallas import tpu as pltpu
from jax.experimental.pallas import tpu_sc as plsc

_TM = 128
_NC, _NS = 2, 16
_NW = _NC * _NS
_LANES = 16


def _routing_call(x2, Wg, bg2, sh11, TM, NT):
    T, D = x2.shape
    E = Wg.shape[1]
    T2 = 2 * T

    def body(x_ref, wg_ref, bg_ref, sh_ref, pe_ref, po_ref, ge_ref, go_ref,
             te_ref, nr_ref):
        x = x_ref[...]
        logits = jnp.dot(x, wg_ref[...], preferred_element_type=jnp.float32)
        logits = logits + bg_ref[...]
        m = jnp.max(logits, axis=1, keepdims=True)
        ex = jnp.exp(logits - m)
        s = ex / jnp.sum(ex, axis=1, keepdims=True)
        io = lax.broadcasted_iota(jnp.int32, (T, E), 1)
        v1 = jnp.max(s, axis=1, keepdims=True)
        i1 = jnp.min(jnp.where(s == v1, io, E), axis=1, keepdims=True)
        sm = jnp.where(io == i1, -1.0, s)
        v2 = jnp.max(sm, axis=1, keepdims=True)
        i2 = jnp.min(jnp.where(sm == v2, io, E), axis=1, keepdims=True)
        sh = sh_ref[0, 0]
        e1 = i1 + sh
        e2 = i2 + sh
        g1 = jnp.sum(jnp.where(io == e1, s, 0.0), axis=1, keepdims=True)
        g2 = jnp.sum(jnp.where(io == e2, s, 0.0), axis=1, keepdims=True)
        den = jnp.maximum(g1 + g2, 1e-12)
        ge_ref[...] = g1 / den
        go_ref[...] = g2 / den

        ec = jnp.concatenate([jnp.clip(e1, 0, E - 1), jnp.clip(e2, 0, E - 1)],
                             axis=0)
        io2 = lax.broadcasted_iota(jnp.int32, (T2, E), 1)
        oh = (ec == io2).astype(jnp.int32)
        c = oh
        d = 1
        while d < T2:
            c = c + jnp.concatenate(
                [jnp.zeros((d, E), jnp.int32), c[:T2 - d]], axis=0)
            d *= 2
        rank = jnp.sum(oh * c, axis=1, keepdims=True) - 1
        counts = c[T2 - 1:T2]
        pc = ((counts + (TM - 1)) // TM) * TM
        ic = pc
        d = 1
        while d < E:
            ic = ic + jnp.concatenate(
                [jnp.zeros((1, d), jnp.int32), ic[:, :E - d]], axis=1)
            d *= 2
        off = ic - pc
        pos = jnp.sum(oh * off, axis=1, keepdims=True) + rank
        pe_ref[...] = pos[:T]
        po_ref[...] = pos[T:]
        nr_ref[...] = ic[:, E - 1:E] // TM
        starts = lax.broadcasted_iota(jnp.int32, (NT, E), 0) * TM
        te_ref[...] = jnp.sum((off <= starts).astype(jnp.int32),
                              axis=1, keepdims=True) - 1

    return pl.pallas_call(
        body,
        out_shape=[jax.ShapeDtypeStruct((T, 1), jnp.int32),
                   jax.ShapeDtypeStruct((T, 1), jnp.int32),
                   jax.ShapeDtypeStruct((T, 1), jnp.float32),
                   jax.ShapeDtypeStruct((T, 1), jnp.float32),
                   jax.ShapeDtypeStruct((NT, 1), jnp.int32),
                   jax.ShapeDtypeStruct((1, 1), jnp.int32)],
    )(x2, Wg, bg2, sh11)


def _ffn_call(te, nr, xs, W1, b1, W2, b2, NT, TM):
    E, D, H = W1.shape
    O = W2.shape[2]
    P = xs.shape[0]

    def body(te_ref, nr_ref, x_ref, w1_ref, b1_ref, w2_ref, b2_ref, o_ref):
        @pl.when(pl.program_id(0) < nr_ref[0])
        def _():
            h = jnp.dot(x_ref[...], w1_ref[0],
                        preferred_element_type=jnp.float32)
            h = jnp.maximum(h + b1_ref[0], 0.0)
            y = jnp.dot(h, w2_ref[0],
                        preferred_element_type=jnp.float32) + b2_ref[0]
            yb = y.astype(jnp.bfloat16)
            lo = lax.bitcast_convert_type(yb[:, :O // 2], jnp.uint16)
            hi = lax.bitcast_convert_type(yb[:, O // 2:], jnp.uint16)
            word = lo.astype(jnp.int32) | (hi.astype(jnp.int32) << 16)
            o_ref[...] = word

    grid_spec = pltpu.PrefetchScalarGridSpec(
        num_scalar_prefetch=2,
        grid=(NT,),
        in_specs=[
            pl.BlockSpec((TM, D), lambda m, te, nr: (m, 0)),
            pl.BlockSpec((1, D, H), lambda m, te, nr: (te[m], 0, 0)),
            pl.BlockSpec((1, 1, H), lambda m, te, nr: (te[m], 0, 0)),
            pl.BlockSpec((1, H, O), lambda m, te, nr: (te[m], 0, 0)),
            pl.BlockSpec((1, 1, O), lambda m, te, nr: (te[m], 0, 0)),
        ],
        out_specs=pl.BlockSpec((TM, O // 2), lambda m, te, nr: (m, 0)),
    )
    return pl.pallas_call(
        body,
        grid_spec=grid_spec,
        out_shape=jax.ShapeDtypeStruct((P, O // 2), jnp.int32),
        compiler_params=pltpu.CompilerParams(
            dimension_semantics=("arbitrary",)),
    )(te, nr, xs, W1, b1, W2, b2)


def _sc_scatter(x2, pe, po, P):
    T, D = x2.shape
    tpw = T // _NW
    mesh = plsc.VectorSubcoreMesh(core_axis_name="c", subcore_axis_name="s",
                                  num_cores=_NC, num_subcores=_NS)

    @functools.partial(
        pl.kernel,
        out_type=jax.ShapeDtypeStruct((P, D), x2.dtype),
        mesh=mesh,
        scratch_types=[pltpu.VMEM((tpw,), jnp.int32),
                       pltpu.VMEM((tpw,), jnp.int32),
                       pltpu.VMEM((tpw, D), x2.dtype),
                       pltpu.SemaphoreType.DMA],
    )
    def k(x_hbm, pe_hbm, po_hbm, out_hbm, ie_v, io_v, rows_v, sem):
        wid = lax.axis_index("s") * _NC + lax.axis_index("c")
        base = wid * tpw
        c0 = pltpu.async_copy(pe_hbm.at[pl.ds(base, tpw)], ie_v, sem)
        c0b = pltpu.async_copy(po_hbm.at[pl.ds(base, tpw)], io_v, sem)
        c0c = pltpu.async_copy(x_hbm.at[pl.ds(base, tpw)], rows_v, sem)
        c0.wait()
        c0b.wait()
        c0c.wait()
        c1 = pltpu.async_copy(rows_v, out_hbm.at[ie_v], sem)
        c2 = pltpu.async_copy(rows_v, out_hbm.at[io_v], sem)
        c1.wait()
        c2.wait()

    return k(x2, pe, po)


def _sc_combine(ys, pe, po, ge, go, T, O):
    W = ys.shape[1]
    tpw = T // _NW
    ch = 16
    nch = tpw // ch
    mesh = plsc.VectorSubcoreMesh(core_axis_name="c", subcore_axis_name="s",
                                  num_cores=_NC, num_subcores=_NS)

    @functools.partial(
        pl.kernel,
        out_type=jax.ShapeDtypeStruct((T, O), jnp.float32),
        mesh=mesh,
        scratch_types=[pltpu.VMEM((tpw,), jnp.int32),
                       pltpu.VMEM((tpw,), jnp.int32),
                       pltpu.VMEM((tpw,), jnp.float32),
                       pltpu.VMEM((tpw,), jnp.float32),
                       pltpu.VMEM((2, ch, W), jnp.int32),
                       pltpu.VMEM((2, ch, W), jnp.int32),
                       pltpu.VMEM((2, ch, O), jnp.float32),
                       pltpu.SemaphoreType.DMA,
                       pltpu.SemaphoreType.DMA,
                       pltpu.SemaphoreType.DMA],
        compiler_params=pltpu.CompilerParams(needs_layout_passes=False),
    )
    def k(ys_hbm, pe_hbm, po_hbm, ge_hbm, go_hbm, out_hbm,
          pe_s, po_s, ge_s, go_s, a_v, b_v, o_v, sg0, sg1, ss):
        wid = lax.axis_index("s") * _NC + lax.axis_index("c")
        base = wid * tpw
        pltpu.sync_copy(pe_hbm.at[pl.ds(base, tpw)], pe_s)
        pltpu.sync_copy(po_hbm.at[pl.ds(base, tpw)], po_s)
        pltpu.sync_copy(ge_hbm.at[pl.ds(base, tpw)], ge_s)
        pltpu.sync_copy(go_hbm.at[pl.ds(base, tpw)], go_s)
        sgs = (sg0, sg1)

        def fire(c):
            buf = c % 2
            iv_e = pe_s[pl.ds(c * ch, ch)]
            iv_o = po_s[pl.ds(c * ch, ch)]
            d1 = pltpu.async_copy(ys_hbm.at[iv_e], a_v.at[buf], sgs[buf])
            d2 = pltpu.async_copy(ys_hbm.at[iv_o], b_v.at[buf], sgs[buf])
            return d1, d2

        descs = {0: fire(0)}
        stores = []
        for c in range(nch):
            buf = c % 2
            d1, d2 = descs.pop(c)
            d1.wait()
            d2.wait()
            if c + 1 < nch:
                if c >= 1:
                    stores[c - 1].wait()
                descs[c + 1] = fire(c + 1)

            def rowgrp(g, carry):
                r0 = g * 4
                for j in range(4):
                    r = r0 + j
                    ridx = jnp.full((_LANES,), c * ch + r, jnp.int32)
                    ga = plsc.load_gather(ge_s, [ridx])
                    gb = plsc.load_gather(go_s, [ridx])
                    gab = plsc.pack(ga, ga,
                                    format=plsc.PackFormat.INTERLEAVED)
                    gbb = plsc.pack(gb, gb,
                                    format=plsc.PackFormat.INTERLEAVED)
                    for cc in range(W // _LANES):
                        sl = pl.ds(cc * _LANES, _LANES)
                        aw = plsc.bitcast(a_v[buf, r, sl], jnp.bfloat16)
                        bw = plsc.bitcast(b_v[buf, r, sl], jnp.bfloat16)
                        rr = gab * aw + gbb * bw
                        u0, u1 = plsc.unpack(
                            rr, format=plsc.PackFormat.INTERLEAVED)
                        o_v[buf, r, sl] = u0
                        o_v[buf, r, pl.ds(O // 2 + cc * _LANES, _LANES)] = u1
                return carry

            lax.fori_loop(0, ch // 4, rowgrp, 0)
            stores.append(pltpu.async_copy(
                o_v.at[buf], out_hbm.at[pl.ds(base + c * ch, ch)], ss))
        stores[-2].wait()
        stores[-1].wait()

    return k(ys, pe, po, ge, go)


def kernel(x, Wg, bg, W1, b1, W2, b2, num_experts_per_token):
    B, T, D = x.shape
    E = Wg.shape[1]
    O = W2.shape[2]
    TM = _TM
    P = 2 * T + E * TM
    NT = P // TM
    x2 = x.reshape(T, D)
    sh11 = (jnp.asarray(num_experts_per_token, jnp.int32) - 2).reshape(1, 1)
    pe, po, ge, go, te, nr = _routing_call(x2, Wg, bg.reshape(1, E), sh11,
                                           TM, NT)
    pe = pe.reshape(T)
    po = po.reshape(T)
    ge = ge.reshape(T)
    go = go.reshape(T)
    xs = _sc_scatter(x2, pe, po, P)
    ys = _ffn_call(te.reshape(NT), nr.reshape(1), xs, W1,
                   b1.reshape(E, 1, -1), W2, b2.reshape(E, 1, -1), NT, TM)
    out = _sc_combine(ys, pe, po, ge, go, T, O)
    return out.reshape(B, T, O)


# speedup vs baseline: 1.0274x; 1.0274x over previous
import functools

import jax
import jax.numpy as jnp
from jax import lax
from jax.experimental import pallas as pl
from jax.experimental.pallas import tpu as pltpu
from jax.experimental.pallas import tpu_sc as plsc

_TM = 128
_NC, _NS = 2, 16
_NW = _NC * _NS
_LANES = 16


def _routing_call(x2, Wg, bg2, sh11, TM, NT):
    T, D = x2.shape
    E = Wg.shape[1]
    T2 = 2 * T

    def body(x_ref, wg_ref, bg_ref, sh_ref, pe_ref, po_ref, ge_ref, go_ref,
             te_ref, nr_ref):
        x = x_ref[...]
        logits = jnp.dot(x, wg_ref[...], preferred_element_type=jnp.float32)
        logits = logits + bg_ref[...]
        m = jnp.max(logits, axis=1, keepdims=True)
        ex = jnp.exp(logits - m)
        s = ex / jnp.sum(ex, axis=1, keepdims=True)
        io = lax.broadcasted_iota(jnp.int32, (T, E), 1)
        v1 = jnp.max(s, axis=1, keepdims=True)
        i1 = jnp.min(jnp.where(s == v1, io, E), axis=1, keepdims=True)
        sm = jnp.where(io == i1, -1.0, s)
        v2 = jnp.max(sm, axis=1, keepdims=True)
        i2 = jnp.min(jnp.where(sm == v2, io, E), axis=1, keepdims=True)
        sh = sh_ref[0, 0]
        e1 = i1 + sh
        e2 = i2 + sh
        g1 = jnp.sum(jnp.where(io == e1, s, 0.0), axis=1, keepdims=True)
        g2 = jnp.sum(jnp.where(io == e2, s, 0.0), axis=1, keepdims=True)
        den = jnp.maximum(g1 + g2, 1e-12)
        ge_ref[...] = g1 / den
        go_ref[...] = g2 / den

        ec = jnp.concatenate([jnp.clip(e1, 0, E - 1), jnp.clip(e2, 0, E - 1)],
                             axis=0)
        io2 = lax.broadcasted_iota(jnp.int32, (T2, E), 1)
        oh = (ec == io2).astype(jnp.int32)
        c = oh
        d = 1
        while d < T2:
            c = c + jnp.concatenate(
                [jnp.zeros((d, E), jnp.int32), c[:T2 - d]], axis=0)
            d *= 2
        rank = jnp.sum(oh * c, axis=1, keepdims=True) - 1
        counts = c[T2 - 1:T2]
        pc = ((counts + (TM - 1)) // TM) * TM
        ic = pc
        d = 1
        while d < E:
            ic = ic + jnp.concatenate(
                [jnp.zeros((1, d), jnp.int32), ic[:, :E - d]], axis=1)
            d *= 2
        off = ic - pc
        pos = jnp.sum(oh * off, axis=1, keepdims=True) + rank
        pe_ref[...] = pos[:T]
        po_ref[...] = pos[T:]
        nr_ref[...] = ic[:, E - 1:E] // TM
        starts = lax.broadcasted_iota(jnp.int32, (NT, E), 0) * TM
        te_ref[...] = jnp.sum((off <= starts).astype(jnp.int32),
                              axis=1, keepdims=True) - 1

    return pl.pallas_call(
        body,
        out_shape=[jax.ShapeDtypeStruct((T, 1), jnp.int32),
                   jax.ShapeDtypeStruct((T, 1), jnp.int32),
                   jax.ShapeDtypeStruct((T, 1), jnp.float32),
                   jax.ShapeDtypeStruct((T, 1), jnp.float32),
                   jax.ShapeDtypeStruct((NT, 1), jnp.int32),
                   jax.ShapeDtypeStruct((1, 1), jnp.int32)],
    )(x2, Wg, bg2, sh11)


def _ffn_call(te, nr, xs, W1, b1, W2, b2, NT, TM):
    E, D, H = W1.shape
    O = W2.shape[2]
    P = xs.shape[0]

    def body(te_ref, nr_ref, x_ref, w1_ref, b1_ref, w2_ref, b2_ref, o_ref):
        @pl.when(pl.program_id(0) < nr_ref[0])
        def _():
            h = jnp.dot(x_ref[...], w1_ref[0],
                        preferred_element_type=jnp.float32)
            h = jnp.maximum(h + b1_ref[0], 0.0)
            o_ref[...] = jnp.dot(h, w2_ref[0],
                                 preferred_element_type=jnp.float32) + b2_ref[0]

    grid_spec = pltpu.PrefetchScalarGridSpec(
        num_scalar_prefetch=2,
        grid=(NT,),
        in_specs=[
            pl.BlockSpec((TM, D), lambda m, te, nr: (m, 0)),
            pl.BlockSpec((1, D, H), lambda m, te, nr: (te[m], 0, 0)),
            pl.BlockSpec((1, 1, H), lambda m, te, nr: (te[m], 0, 0)),
            pl.BlockSpec((1, H, O), lambda m, te, nr: (te[m], 0, 0)),
            pl.BlockSpec((1, 1, O), lambda m, te, nr: (te[m], 0, 0)),
        ],
        out_specs=pl.BlockSpec((TM, O), lambda m, te, nr: (m, 0)),
    )
    return pl.pallas_call(
        body,
        grid_spec=grid_spec,
        out_shape=jax.ShapeDtypeStruct((P, O), jnp.float32),
        compiler_params=pltpu.CompilerParams(
            dimension_semantics=("arbitrary",)),
    )(te, nr, xs, W1, b1, W2, b2)


def _sc_scatter(x2, pe, po, P):
    T, D = x2.shape
    tpw = T // _NW
    mesh = plsc.VectorSubcoreMesh(core_axis_name="c", subcore_axis_name="s",
                                  num_cores=_NC, num_subcores=_NS)

    @functools.partial(
        pl.kernel,
        out_type=jax.ShapeDtypeStruct((P, D), x2.dtype),
        mesh=mesh,
        scratch_types=[pltpu.VMEM((tpw,), jnp.int32),
                       pltpu.VMEM((tpw,), jnp.int32),
                       pltpu.VMEM((tpw, D), x2.dtype),
                       pltpu.SemaphoreType.DMA],
    )
    def k(x_hbm, pe_hbm, po_hbm, out_hbm, ie_v, io_v, rows_v, sem):
        wid = lax.axis_index("s") * _NC + lax.axis_index("c")
        base = wid * tpw
        c0 = pltpu.async_copy(pe_hbm.at[pl.ds(base, tpw)], ie_v, sem)
        c0b = pltpu.async_copy(po_hbm.at[pl.ds(base, tpw)], io_v, sem)
        c0c = pltpu.async_copy(x_hbm.at[pl.ds(base, tpw)], rows_v, sem)
        c0.wait()
        c0b.wait()
        c0c.wait()
        c1 = pltpu.async_copy(rows_v, out_hbm.at[ie_v], sem)
        c2 = pltpu.async_copy(rows_v, out_hbm.at[io_v], sem)
        c1.wait()
        c2.wait()

    return k(x2, pe, po)


def _sc_combine(ys, pe, po, ge, go, T):
    O = ys.shape[1]
    tpw = T // _NW
    ch = 16
    nch = tpw // ch
    mesh = plsc.VectorSubcoreMesh(core_axis_name="c", subcore_axis_name="s",
                                  num_cores=_NC, num_subcores=_NS)

    @functools.partial(
        pl.kernel,
        out_type=jax.ShapeDtypeStruct((T, O), jnp.float32),
        mesh=mesh,
        scratch_types=[pltpu.VMEM((tpw,), jnp.int32),
                       pltpu.VMEM((tpw,), jnp.int32),
                       pltpu.VMEM((tpw,), jnp.float32),
                       pltpu.VMEM((tpw,), jnp.float32),
                       pltpu.VMEM((2, ch, O), jnp.float32),
                       pltpu.VMEM((2, ch, O), jnp.float32),
                       pltpu.SemaphoreType.DMA,
                       pltpu.SemaphoreType.DMA,
                       pltpu.SemaphoreType.DMA],
        compiler_params=pltpu.CompilerParams(needs_layout_passes=False),
    )
    def k(ys_hbm, pe_hbm, po_hbm, ge_hbm, go_hbm, out_hbm,
          pe_s, po_s, ge_s, go_s, a_v, b_v, sg0, sg1, ss):
        wid = lax.axis_index("s") * _NC + lax.axis_index("c")
        base = wid * tpw
        pltpu.sync_copy(pe_hbm.at[pl.ds(base, tpw)], pe_s)
        pltpu.sync_copy(po_hbm.at[pl.ds(base, tpw)], po_s)
        pltpu.sync_copy(ge_hbm.at[pl.ds(base, tpw)], ge_s)
        pltpu.sync_copy(go_hbm.at[pl.ds(base, tpw)], go_s)
        sgs = (sg0, sg1)

        def fire(c):
            buf = c % 2
            iv_e = pe_s[pl.ds(c * ch, ch)]
            iv_o = po_s[pl.ds(c * ch, ch)]
            d1 = pltpu.async_copy(ys_hbm.at[iv_e], a_v.at[buf], sgs[buf])
            d2 = pltpu.async_copy(ys_hbm.at[iv_o], b_v.at[buf], sgs[buf])
            return d1, d2

        descs = {0: fire(0)}
        stores = []
        for c in range(nch):
            buf = c % 2
            d1, d2 = descs.pop(c)
            d1.wait()
            d2.wait()
            if c + 1 < nch:
                if c >= 1:
                    stores[c - 1].wait()
                descs[c + 1] = fire(c + 1)

            def row(r, carry):
                ridx = jnp.full((_LANES,), c * ch + r, jnp.int32)
                ga = plsc.load_gather(ge_s, [ridx])
                gb = plsc.load_gather(go_s, [ridx])
                for cc in range(O // _LANES):
                    sl = pl.ds(cc * _LANES, _LANES)
                    a_v[buf, r, sl] = (ga * a_v[buf, r, sl]
                                       + gb * b_v[buf, r, sl])
                return carry

            lax.fori_loop(0, ch, row, 0)
            stores.append(pltpu.async_copy(
                a_v.at[buf], out_hbm.at[pl.ds(base + c * ch, ch)], ss))
        stores[-2].wait()
        stores[-1].wait()

    return k(ys, pe, po, ge, go)


def kernel(x, Wg, bg, W1, b1, W2, b2, num_experts_per_token):
    B, T, D = x.shape
    E = Wg.shape[1]
    O = W2.shape[2]
    TM = _TM
    P = 2 * T + E * TM
    NT = P // TM
    x2 = x.reshape(T, D)
    sh11 = (jnp.asarray(num_experts_per_token, jnp.int32) - 2).reshape(1, 1)
    pe, po, ge, go, te, nr = _routing_call(x2, Wg, bg.reshape(1, E), sh11,
                                           TM, NT)
    pe = pe.reshape(T)
    po = po.reshape(T)
    ge = ge.reshape(T)
    go = go.reshape(T)
    xs = _sc_scatter(x2, pe, po, P)
    ys = _ffn_call(te.reshape(NT), nr.reshape(1), xs, W1,
                   b1.reshape(E, 1, -1), W2, b2.reshape(E, 1, -1), NT, TM)
    out = _sc_combine(ys, pe, po, ge, go, T)
    return out.reshape(B, T, O)
